# baseline (speedup 1.0000x reference)
import functools

import jax
import jax.numpy as jnp
from jax.experimental import pallas as pl
from jax.experimental.pallas import tpu as pltpu

_POOL = 8


def _chs_kernel(gt_ref, om0_ref, om1_ref, w_ref, out_ref,
                bits_ref, delta_ref, acc_ref, *,
                rows_in, cols_in, rows_out, cols_out, n_chunks, num):
    b = pl.program_id(0)
    j = pl.program_id(1)

    @pl.when((b == 0) & (j == 0))
    def _init():
        acc_ref[0] = 0.0

    x = gt_ref[0, 0]
    io = jax.lax.broadcasted_iota
    ph = (io(jnp.int32, (rows_out, rows_in), 1) // _POOL
          == io(jnp.int32, (rows_out, rows_in), 0)).astype(jnp.float32)
    xh = jnp.dot(ph, x, preferred_element_type=jnp.float32)
    pw = (io(jnp.int32, (cols_in, cols_out), 0) // _POOL
          == io(jnp.int32, (cols_in, cols_out), 1)).astype(jnp.float32)
    dmap = jnp.dot(xh, pw, preferred_element_type=jnp.float32)

    om0 = om0_ref[0]
    om1 = om1_ref[0]
    w = w_ref[0]
    d_base = om0 - dmap
    err = jnp.abs(d_base)
    bits_ref[j] = jax.lax.bitcast_convert_type(err, jnp.int32)
    d_comb = om0 - (w * om1 + (1.0 - w) * dmap)
    base = d_base * d_base
    delta_ref[j] = d_comb * d_comb - base
    acc_ref[0] += jnp.sum(base)

    @pl.when(j == n_chunks - 1)
    def _finish_row():
        bits = bits_ref[...]

        def body(i, res):
            cand = res | (jnp.int32(1) << (jnp.int32(30) - i))
            cnt = jnp.sum((bits >= cand).astype(jnp.int32))
            return jnp.where(cnt >= num, cand, res)

        thr = jax.lax.fori_loop(0, 31, body, jnp.int32(0))
        corr = jnp.sum(jnp.where(bits >= thr, delta_ref[...], 0.0))
        acc_ref[0] += corr

    @pl.when((b == pl.num_programs(0) - 1) & (j == n_chunks - 1))
    def _emit():
        out_ref[...] = jnp.full((1, 1), acc_ref[0], jnp.float32)


def kernel(output_map_0, output_map_1, gt_density, process):
    b, c, h, w = output_map_0.shape
    B, C, H, W = gt_density.shape
    num = int(h * w * 0.1)

    rows_in = 1024
    rows_out = rows_in // _POOL
    n_chunks = H // rows_in

    om0 = output_map_0.reshape(B, h, w)
    om1 = output_map_1.reshape(B, h, w)
    wmat = jnp.broadcast_to(jnp.asarray(process, jnp.float32), (1, 1, 1))

    loss = pl.pallas_call(
        functools.partial(_chs_kernel, rows_in=rows_in, cols_in=W,
                          rows_out=rows_out, cols_out=w,
                          n_chunks=n_chunks, num=num),
        grid=(B, n_chunks),
        in_specs=[
            pl.BlockSpec((1, 1, rows_in, W), lambda bi, j: (bi, 0, j, 0)),
            pl.BlockSpec((1, rows_out, w), lambda bi, j: (bi, j, 0)),
            pl.BlockSpec((1, rows_out, w), lambda bi, j: (bi, j, 0)),
            pl.BlockSpec((1, 1, 1), lambda bi, j: (0, 0, 0)),
        ],
        out_specs=pl.BlockSpec((1, 1), lambda bi, j: (0, 0)),
        out_shape=jax.ShapeDtypeStruct((1, 1), jnp.float32),
        scratch_shapes=[
            pltpu.VMEM((n_chunks, rows_out, w), jnp.int32),
            pltpu.VMEM((n_chunks, rows_out, w), jnp.float32),
            pltpu.SMEM((1,), jnp.float32),
        ],
    )(gt_density, om0, om1, wmat)
    return loss[0, 0]
